# Initial kernel scaffold; baseline (speedup 1.0000x reference)
import jax
import jax.numpy as jnp
from jax.experimental import pallas as pl


def kernel(probs, index, latent):
    raise NotImplementedError("write your pallas kernel here")


# speedup vs baseline: 1.7940x; 1.7940x over previous
import functools

import jax
import jax.numpy as jnp
from jax import lax
from jax.experimental import pallas as pl
from jax.experimental.pallas import tpu as pltpu
from jax.experimental.pallas import tpu_sc as plsc

_BATCH = 16384
_D = 64
_BETA = 0.9
_NC = 2
_NS = 16
_NW = _NC * _NS
_BPW = _BATCH // _NW
_CH = 128
_NCH = _BPW // _CH
_L = 16
_NSL = _D // _L
_NG = _BPW // _L
_LO = 0.0001
_HI = 1.0 - 0.0001


@functools.partial(
    pl.kernel,
    mesh=plsc.VectorSubcoreMesh(core_axis_name="c", subcore_axis_name="s"),
    scratch_types=[
        pltpu.VMEM((_NCH, _CH), jnp.int32),
        pltpu.VMEM((_BPW, _D), jnp.float32),
        pltpu.VMEM((_D, _BPW), jnp.float32),
        pltpu.VMEM((_BPW, _D), jnp.float32),
        pltpu.SemaphoreType.DMA,
    ],
    compiler_params=pltpu.CompilerParams(use_tc_tiling_on_sc=False),
)
def _sc_ema_scatter(
    probs_hbm, probs_t_hbm, idx_hbm, latent_ref,
    idx_v, probs_v, probs_tv, rows_v, sem,
):
    wid = lax.axis_index("s") * _NC + lax.axis_index("c")
    base = wid * _BPW

    pltpu.sync_copy(idx_hbm.at[wid], idx_v)
    gathers = [
        pltpu.async_copy(
            latent_ref.at[idx_v.at[j]],
            rows_v.at[pl.ds(j * _CH, _CH)],
            sem,
        )
        for j in range(_NCH)
    ]
    pltpu.sync_copy(probs_hbm.at[pl.ds(base, _BPW)], probs_v)
    pltpu.sync_copy(probs_t_hbm.at[:, pl.ds(base, _BPW)], probs_tv)

    for cp in gathers:
        cp.wait()

    def group_body(g, carry):
        acc = None
        for c in range(_D):
            v = probs_tv[c, pl.ds(g * _L, _L)]
            v = jnp.minimum(jnp.maximum(v, _LO), _HI)
            acc = v if acc is None else acc + v
        sv = (1.0 - _BETA) / acc
        for k in range(_L):
            r = g * _L + k
            s = sv[k]
            for j in range(_NSL):
                p = probs_v[r, pl.ds(j * _L, _L)]
                p = jnp.minimum(jnp.maximum(p, _LO), _HI)
                gr = rows_v[r, pl.ds(j * _L, _L)]
                rows_v[r, pl.ds(j * _L, _L)] = _BETA * gr + p * s
        return carry

    lax.fori_loop(0, _NG, group_body, 0)

    scatters = [
        pltpu.async_copy(
            rows_v.at[pl.ds(j * _CH, _CH)],
            latent_ref.at[idx_v.at[j]],
            sem,
        )
        for j in range(_NCH)
    ]
    for cp in scatters:
        cp.wait()


def kernel(probs, index, latent):
    idx3 = index.reshape(_NW, _NCH, _CH)
    probs_t = probs.T
    out_ref = jax.new_ref(latent)
    _sc_ema_scatter(probs, probs_t, idx3, out_ref)
    return out_ref[...]
